# Initial kernel scaffold; baseline (speedup 1.0000x reference)
import jax
import jax.numpy as jnp
from jax.experimental import pallas as pl


def kernel(x_nchw, w0, w1, w2):
    raise NotImplementedError("write your pallas kernel here")


# speedup vs baseline: 1.1709x; 1.1709x over previous
import jax
import jax.numpy as jnp
from jax.experimental import pallas as pl
from jax.experimental.pallas import tpu as pltpu
from jax.scipy.linalg import block_diag

_LEAK = 0.1
_GROUP = 8
_BTILE = 32768


def _fused_mlp_kernel(x_ref, w0_ref, w1_ref, w2_ref, o_ref):
    h = jnp.dot(x_ref[...], w0_ref[...], preferred_element_type=jnp.float32)
    h = jnp.maximum(h, _LEAK * h)
    h = jnp.dot(h, w1_ref[...], preferred_element_type=jnp.float32)
    h = jnp.maximum(h, _LEAK * h)
    o_ref[...] = jnp.dot(h, w2_ref[...], preferred_element_type=jnp.float32)


def _round_up(n, m):
    return -(-n // m) * m


def kernel(x_nchw, w0, w1, w2):
    B, c_in = x_nchw.shape[0], x_nchw.shape[1]
    c_out = w2.shape[0]
    g = _GROUP

    w0t = w0.reshape(w0.shape[0], c_in).T
    w1t = w1.reshape(w1.shape[0], w1.shape[1]).T
    w2t = w2.reshape(c_out, w2.shape[1]).T
    w0bd = block_diag(*([w0t] * g))
    w1bd = block_diag(*([w1t] * g))
    w2bd = block_diag(*([w2t] * g))

    btile = min(_BTILE, _round_up(B, g))
    bpad = _round_up(B, btile)
    x = x_nchw.reshape(B, c_in)
    if bpad != B:
        x = jnp.pad(x, ((0, bpad - B), (0, 0)))
    xg = x.reshape(bpad // g, g * c_in)

    grid = (bpad // btile,)
    rows = btile // g

    out = pl.pallas_call(
        _fused_mlp_kernel,
        out_shape=jax.ShapeDtypeStruct((bpad // g, g), jnp.float32),
        grid=grid,
        in_specs=[
            pl.BlockSpec((rows, g * c_in), lambda i: (i, 0)),
            pl.BlockSpec(w0bd.shape, lambda i: (0, 0)),
            pl.BlockSpec(w1bd.shape, lambda i: (0, 0)),
            pl.BlockSpec(w2bd.shape, lambda i: (0, 0)),
        ],
        out_specs=pl.BlockSpec((rows, g), lambda i: (i, 0)),
        compiler_params=pltpu.CompilerParams(
            dimension_semantics=("parallel",),
            vmem_limit_bytes=64 * 1024 * 1024,
        ),
    )(xg, w0bd, w1bd, w2bd)

    return out.reshape(bpad)[:B].reshape(B, c_out, 1, 1)


# speedup vs baseline: 1.2452x; 1.0634x over previous
import jax
import jax.numpy as jnp
from jax.experimental import pallas as pl
from jax.experimental.pallas import tpu as pltpu

_LEAK = 0.1
_BTILE = 8192
_OUT_PAD = 8


def _fused_mlp_kernel(x_ref, w0_ref, w1_ref, w2_ref, o_ref):
    h = jnp.dot(x_ref[...], w0_ref[...], preferred_element_type=jnp.float32)
    h = jnp.maximum(h, _LEAK * h)
    h = jnp.dot(h, w1_ref[...], preferred_element_type=jnp.float32)
    h = jnp.maximum(h, _LEAK * h)
    o_ref[...] = jnp.dot(h, w2_ref[...], preferred_element_type=jnp.float32)


def _round_up(n, m):
    return -(-n // m) * m


def kernel(x_nchw, w0, w1, w2):
    B, c_in = x_nchw.shape[0], x_nchw.shape[1]
    c_out = w2.shape[0]

    x = x_nchw.reshape(B, c_in)
    w0t = w0.reshape(w0.shape[0], c_in).T
    w1t = w1.reshape(w1.shape[0], w1.shape[1]).T
    w2t = w2.reshape(c_out, w2.shape[1]).T
    w2t = jnp.pad(w2t, ((0, 0), (0, _OUT_PAD - c_out)))

    btile = min(_BTILE, _round_up(B, 8))
    bpad = _round_up(B, btile)
    if bpad != B:
        x = jnp.pad(x, ((0, bpad - B), (0, 0)))
    grid = (bpad // btile,)

    out = pl.pallas_call(
        _fused_mlp_kernel,
        out_shape=jax.ShapeDtypeStruct((bpad, _OUT_PAD), jnp.float32),
        grid=grid,
        in_specs=[
            pl.BlockSpec((btile, c_in), lambda i: (i, 0)),
            pl.BlockSpec(w0t.shape, lambda i: (0, 0)),
            pl.BlockSpec(w1t.shape, lambda i: (0, 0)),
            pl.BlockSpec(w2t.shape, lambda i: (0, 0)),
        ],
        out_specs=pl.BlockSpec((btile, _OUT_PAD), lambda i: (i, 0)),
        compiler_params=pltpu.CompilerParams(
            dimension_semantics=("parallel",),
            vmem_limit_bytes=64 * 1024 * 1024,
        ),
    )(x, w0t, w1t, w2t)

    return out[:B, :c_out].reshape(B, c_out, 1, 1)


# speedup vs baseline: 3.0098x; 2.4171x over previous
import jax
import jax.numpy as jnp
from jax.experimental import pallas as pl
from jax.experimental.pallas import tpu as pltpu

_LEAK = 0.1
_LTILE = 32768
_SUBLANE = 8


def _fused_mlp_kernel(x_ref, w0_ref, w1_ref, w2_ref, o_ref):
    h = jnp.dot(w0_ref[...], x_ref[...], preferred_element_type=jnp.float32)
    h = jnp.maximum(h, _LEAK * h)
    h = jnp.dot(w1_ref[...], h, preferred_element_type=jnp.float32)
    h = jnp.maximum(h, _LEAK * h)
    o_ref[...] = jnp.dot(w2_ref[...], h, preferred_element_type=jnp.float32)


def _round_up(n, m):
    return -(-n // m) * m


def kernel(x_nchw, w0, w1, w2):
    B, c_in = x_nchw.shape[0], x_nchw.shape[1]
    c_out = w2.shape[0]

    xt = x_nchw.reshape(B, c_in).T
    w0m = w0.reshape(w0.shape[0], c_in)
    w1m = w1.reshape(w1.shape[0], w1.shape[1])
    w2m = w2.reshape(c_out, w2.shape[1])
    w2p = jnp.pad(w2m, ((0, _SUBLANE - c_out), (0, 0)))

    ltile = min(_LTILE, _round_up(B, 128))
    bpad = _round_up(B, ltile)
    if bpad != B:
        xt = jnp.pad(xt, ((0, 0), (0, bpad - B)))
    grid = (bpad // ltile,)

    out = pl.pallas_call(
        _fused_mlp_kernel,
        out_shape=jax.ShapeDtypeStruct((_SUBLANE, bpad), jnp.float32),
        grid=grid,
        in_specs=[
            pl.BlockSpec((c_in, ltile), lambda i: (0, i)),
            pl.BlockSpec(w0m.shape, lambda i: (0, 0)),
            pl.BlockSpec(w1m.shape, lambda i: (0, 0)),
            pl.BlockSpec(w2p.shape, lambda i: (0, 0)),
        ],
        out_specs=pl.BlockSpec((_SUBLANE, ltile), lambda i: (0, i)),
        compiler_params=pltpu.CompilerParams(
            dimension_semantics=("parallel",),
            vmem_limit_bytes=64 * 1024 * 1024,
        ),
    )(xt, w0m, w1m, w2p)

    return out[:c_out, :B].T.reshape(B, c_out, 1, 1)


# speedup vs baseline: 3.2036x; 1.0644x over previous
import jax
import jax.numpy as jnp
from jax.experimental import pallas as pl
from jax.experimental.pallas import tpu as pltpu

_LEAK = 0.1
_LTILE = 32768
_SUBLANE = 8


def _fused_mlp_kernel(x_ref, w0_ref, w1_ref, w2_ref, o_ref):
    bf = jnp.bfloat16
    leak = jnp.bfloat16(_LEAK)
    x = x_ref[...].astype(bf)
    h = jnp.dot(w0_ref[...].astype(bf), x,
                preferred_element_type=jnp.float32).astype(bf)
    h = jnp.maximum(h, leak * h)
    h = jnp.dot(w1_ref[...].astype(bf), h,
                preferred_element_type=jnp.float32).astype(bf)
    h = jnp.maximum(h, leak * h)
    o_ref[...] = jnp.dot(w2_ref[...].astype(bf), h,
                         preferred_element_type=jnp.float32)


def _round_up(n, m):
    return -(-n // m) * m


def kernel(x_nchw, w0, w1, w2):
    B, c_in = x_nchw.shape[0], x_nchw.shape[1]
    c_out = w2.shape[0]

    xt = x_nchw.reshape(B, c_in).T
    w0m = w0.reshape(w0.shape[0], c_in)
    w1m = w1.reshape(w1.shape[0], w1.shape[1])
    w2m = w2.reshape(c_out, w2.shape[1])

    ltile = min(_LTILE, _round_up(B, 128))
    bpad = _round_up(B, ltile)
    if bpad != B:
        xt = jnp.pad(xt, ((0, 0), (0, bpad - B)))
    grid = (bpad // ltile,)

    out = pl.pallas_call(
        _fused_mlp_kernel,
        out_shape=jax.ShapeDtypeStruct((c_out, bpad), jnp.float32),
        grid=grid,
        in_specs=[
            pl.BlockSpec((c_in, ltile), lambda i: (0, i)),
            pl.BlockSpec(w0m.shape, lambda i: (0, 0)),
            pl.BlockSpec(w1m.shape, lambda i: (0, 0)),
            pl.BlockSpec(w2m.shape, lambda i: (0, 0)),
        ],
        out_specs=pl.BlockSpec((c_out, ltile), lambda i: (0, i)),
        compiler_params=pltpu.CompilerParams(
            dimension_semantics=("parallel",),
            vmem_limit_bytes=64 * 1024 * 1024,
        ),
    )(xt, w0m, w1m, w2m)

    return out[:, :B].reshape(B, c_out, 1, 1)
